# baseline (speedup 1.0000x reference)
import functools

import jax
import jax.numpy as jnp
from jax import lax
from jax.experimental import pallas as pl
from jax.experimental.pallas import tpu as pltpu

_B = 16
_H = 512
_EPS = 1e-5


def _bn_affine(st, g, b, count):
    s = jnp.sum(st, axis=0)
    mean = s[0:1] * (1.0 / count)
    ex2 = s[1:2] * (1.0 / count)
    var = ex2 - mean * mean
    scale = g * lax.rsqrt(var + _EPS)
    shift = b - mean * scale
    return scale, shift


def _stats(y):
    return jnp.concatenate(
        [jnp.sum(y, axis=0, keepdims=True),
         jnp.sum(y * y, axis=0, keepdims=True)], axis=0)


def _conv1_kernel(x_ref, w_ref, b_ref, y_ref, st_ref):
    x = x_ref[0]
    t_out = 396
    y = jnp.dot(x[0:t_out], w_ref[0], preferred_element_type=jnp.float32)
    for k in range(1, 5):
        y = y + jnp.dot(x[k:k + t_out], w_ref[k],
                        preferred_element_type=jnp.float32)
    y = y + b_ref[...]
    y_ref[0] = y
    st_ref[0] = _stats(y)


def _conv_dil_kernel(y_in_ref, st_in_ref, g_ref, bta_ref, w_ref, b_ref,
                     y_ref, st_ref, *, t_in, dil, count):
    scale, shift = _bn_affine(st_in_ref[...], g_ref[...], bta_ref[...], count)
    yn = jnp.clip(y_in_ref[0] * scale + shift, 0.0, 20.0)
    t_out = t_in - 2 * dil
    y = jnp.dot(yn[0:t_out], w_ref[0], preferred_element_type=jnp.float32)
    y = y + jnp.dot(yn[dil:dil + t_out], w_ref[1],
                    preferred_element_type=jnp.float32)
    y = y + jnp.dot(yn[2 * dil:2 * dil + t_out], w_ref[2],
                    preferred_element_type=jnp.float32)
    y = y + b_ref[...]
    y_ref[0] = y
    st_ref[0] = _stats(y)


def _lin_kernel(y_in_ref, st_in_ref, g_ref, bta_ref, w_ref, b_ref,
                y_ref, st_ref, *, count):
    scale, shift = _bn_affine(st_in_ref[...], g_ref[...], bta_ref[...], count)
    xn = jnp.clip(y_in_ref[0] * scale + shift, 0.0, 20.0)
    y = jnp.dot(xn, w_ref[...], preferred_element_type=jnp.float32) + b_ref[...]
    y_ref[0] = y
    st_ref[0] = _stats(y)


def _lin5_kernel(y_in_ref, st_in_ref, g_ref, bta_ref, w_ref, b_ref,
                 x_ref, y_ref, st_ref, *, count):
    scale, shift = _bn_affine(st_in_ref[...], g_ref[...], bta_ref[...], count)
    xn = jnp.clip(y_in_ref[0] * scale + shift, 0.0, 20.0)
    y = jnp.dot(xn, w_ref[...], preferred_element_type=jnp.float32) + b_ref[...]
    x_ref[0] = xn
    y_ref[0] = y
    st_ref[0] = _stats(y)


def _pool_kernel(x_ref, y5_ref, st_in_ref, g_ref, bta_ref, m_ref, *, count):
    scale, shift = _bn_affine(st_in_ref[...], g_ref[...], bta_ref[...], count)
    xb = jnp.clip(y5_ref[0] * scale + shift, 0.0, 20.0)
    m_ref[0] = lax.dot_general(x_ref[0], xb, (((0,), (0,)), ((), ())),
                               preferred_element_type=jnp.float32)


def _bot_kernel(m_ref, w_ref, bb_ref, g_ref, bta_ref, ew_ref, eb_ref,
                out_ref, acc_ref, *, n_steps):
    k = pl.program_id(0)

    @pl.when(k == 0)
    def _():
        acc_ref[...] = jnp.zeros_like(acc_ref)

    acc_ref[...] += lax.dot_general(
        m_ref[...], w_ref[...], (((1,), (1,)), ((), ())),
        preferred_element_type=jnp.float32)

    @pl.when(k == n_steps - 1)
    def _():
        h = acc_ref[...] + bb_ref[...]
        mean = jnp.mean(h, axis=0, keepdims=True)
        var = jnp.mean(h * h, axis=0, keepdims=True) - mean * mean
        hn = jnp.clip(g_ref[...] * (h - mean) * lax.rsqrt(var + _EPS)
                      + bta_ref[...], 0.0, 20.0)
        emb = lax.dot_general(hn, ew_ref[...], (((1,), (1,)), ((), ())),
                              preferred_element_type=jnp.float32)
        emb = emb + eb_ref[...]
        inv = lax.rsqrt(jnp.sum(emb * emb, axis=1, keepdims=True) + 1e-10)
        out_ref[...] = emb * inv * 10.0


def _row(v):
    return v.reshape(1, -1)


def _full(shape):
    return pl.BlockSpec(shape, lambda b: tuple(0 for _ in shape))


def _batched(shape):
    return pl.BlockSpec((1,) + shape, lambda b: (b,) + tuple(0 for _ in shape))


def kernel(input_x, conv1_w, conv1_b, bn1_g, bn1_b, conv2_w, conv2_b, bn2_g,
           bn2_b, conv3_w, conv3_b, bn3_g, bn3_b, lin4_w, lin4_b, bn4_g,
           bn4_b, lin5_w, lin5_b, bn5_g, bn5_b, bot_w, bot_b, bnb_g, bnb_b,
           emb_w, emb_b):
    f32 = jnp.float32
    x = input_x[:, 0]
    b, t0 = x.shape[0], x.shape[1]
    h = conv1_w.shape[0]

    w1 = conv1_w.transpose(2, 1, 0)
    w2 = conv2_w.transpose(2, 1, 0)
    w3 = conv3_w.transpose(2, 1, 0)
    w4 = lin4_w.T
    w5 = lin5_w.T

    par = pltpu.CompilerParams(dimension_semantics=("arbitrary",))

    t1 = t0 - 4
    y1, st1 = pl.pallas_call(
        _conv1_kernel,
        grid=(b,),
        in_specs=[_batched((t0, 40)), _full(w1.shape), _full((1, h))],
        out_specs=[_batched((t1, h)), _batched((2, h))],
        out_shape=[jax.ShapeDtypeStruct((b, t1, h), f32),
                   jax.ShapeDtypeStruct((b, 2, h), f32)],
        compiler_params=par, name="conv1",
    )(x, w1, _row(conv1_b))

    def conv_stage(y_in, st_in, g, beta, w, bias, dil, name):
        t_in = y_in.shape[1]
        t_out = t_in - 2 * dil
        return pl.pallas_call(
            functools.partial(_conv_dil_kernel, t_in=t_in, dil=dil,
                              count=float(b * t_in)),
            grid=(b,),
            in_specs=[_batched((t_in, h)), _full((b, 2, h)), _full((1, h)),
                      _full((1, h)), _full(w.shape), _full((1, h))],
            out_specs=[_batched((t_out, h)), _batched((2, h))],
            out_shape=[jax.ShapeDtypeStruct((b, t_out, h), f32),
                       jax.ShapeDtypeStruct((b, 2, h), f32)],
            compiler_params=par, name=name,
        )(y_in, st_in, _row(g), _row(beta), w, _row(bias))

    y2, st2 = conv_stage(y1, st1, bn1_g, bn1_b, w2, conv2_b, 2, "conv2")
    y3, st3 = conv_stage(y2, st2, bn2_g, bn2_b, w3, conv3_b, 4, "conv3")
    t3 = y3.shape[1]

    y4, st4 = pl.pallas_call(
        functools.partial(_lin_kernel, count=float(b * t3)),
        grid=(b,),
        in_specs=[_batched((t3, h)), _full((b, 2, h)), _full((1, h)),
                  _full((1, h)), _full((h, h)), _full((1, h))],
        out_specs=[_batched((t3, h)), _batched((2, h))],
        out_shape=[jax.ShapeDtypeStruct((b, t3, h), f32),
                   jax.ShapeDtypeStruct((b, 2, h), f32)],
        compiler_params=par, name="lin4",
    )(y3, st3, _row(bn3_g), _row(bn3_b), w4, _row(lin4_b))

    xn, y5, st5 = pl.pallas_call(
        functools.partial(_lin5_kernel, count=float(b * t3)),
        grid=(b,),
        in_specs=[_batched((t3, h)), _full((b, 2, h)), _full((1, h)),
                  _full((1, h)), _full((h, h)), _full((1, h))],
        out_specs=[_batched((t3, h)), _batched((t3, h)), _batched((2, h))],
        out_shape=[jax.ShapeDtypeStruct((b, t3, h), f32),
                   jax.ShapeDtypeStruct((b, t3, h), f32),
                   jax.ShapeDtypeStruct((b, 2, h), f32)],
        compiler_params=par, name="lin5",
    )(y4, st4, _row(bn4_g), _row(bn4_b), w5, _row(lin5_b))

    m = pl.pallas_call(
        functools.partial(_pool_kernel, count=float(b * t3)),
        grid=(b,),
        in_specs=[_batched((t3, h)), _batched((t3, h)), _full((b, 2, h)),
                  _full((1, h)), _full((1, h))],
        out_specs=_batched((h, h)),
        out_shape=jax.ShapeDtypeStruct((b, h, h), f32),
        compiler_params=par, name="pool",
    )(xn, y5, st5, _row(bn5_g), _row(bn5_b))

    msq = bot_w.shape[1]
    m2 = m.reshape(b, msq)
    ck = 4096
    n_steps = msq // ck
    out = pl.pallas_call(
        functools.partial(_bot_kernel, n_steps=n_steps),
        grid=(n_steps,),
        in_specs=[
            pl.BlockSpec((b, ck), lambda k: (0, k)),
            pl.BlockSpec((h, ck), lambda k: (0, k)),
            _full((1, h)), _full((1, h)), _full((1, h)), _full((h, h)),
            _full((1, h)),
        ],
        out_specs=pl.BlockSpec((b, h), lambda k: (0, 0)),
        out_shape=jax.ShapeDtypeStruct((b, h), f32),
        scratch_shapes=[pltpu.VMEM((b, h), f32)],
        compiler_params=pltpu.CompilerParams(
            dimension_semantics=("arbitrary",)),
        name="bot",
    )(m2, bot_w, _row(bot_b), _row(bnb_g), _row(bnb_b), emb_w, _row(emb_b))
    return out
